# baseline (speedup 1.0000x reference)
import jax
import jax.numpy as jnp
from jax import lax
from jax.experimental import pallas as pl
from jax.experimental.pallas import tpu as pltpu
from jax.experimental.pallas import tpu_sc as plsc

NC = 2
NS = 16
NW = NC * NS
L = 16

B = 1024
R = 160
D = 128
BPW = B // NW
SLICE = 80
SPB = R // SLICE
NSL = BPW * SPB
DV = D // L
NRING = 3
K = 2


def _splat(x):
    return jnp.full((L,), x, jnp.int32)


def _body(tok_hbm, gid_hbm, table_hbm, known_hbm, w1t_hbm, b1_hbm, w2t_hbm,
          b2_hbm, br_hbm, out_hbm,
          idx_v, rows_v, w1t_v, w2t_v, known_v, b1_v, b2_v, gid_v,
          br_v, sums_v, h_v, gsem, ssem):
    w = lax.axis_index("s") * NC + lax.axis_index("c")
    pltpu.sync_copy(tok_hbm.at[w], idx_v)
    pltpu.sync_copy(gid_hbm.at[pl.ds(w * BPW, BPW)], gid_v)
    pltpu.sync_copy(known_hbm, known_v)
    pltpu.sync_copy(w1t_hbm, w1t_v)
    pltpu.sync_copy(w2t_hbm, w2t_v)
    pltpu.sync_copy(b1_hbm, b1_v)
    pltpu.sync_copy(b2_hbm, b2_v)
    pltpu.sync_copy(br_hbm, br_v)

    def fire(b):
        buf = b % NRING
        for h in range(SPB):
            pltpu.async_copy(table_hbm.at[idx_v.at[SPB * b + h]],
                             rows_v.at[buf, pl.ds(h * SLICE, SLICE)],
                             gsem.at[buf])

    for b in range(K):
        fire(b)

    @pl.loop(0, BPW)
    def _per_b(b):
        buf = b % NRING
        for h in range(SPB):
            pltpu.make_async_copy(table_hbm.at[idx_v.at[SPB * b + h]],
                                  rows_v.at[buf, pl.ds(h * SLICE, SLICE)],
                                  gsem.at[buf]).wait()

        def acc_row(r, carry):
            return tuple(carry[j] + rows_v[buf, r, pl.ds(j * L, L)]
                         for j in range(DV))

        zero = tuple(jnp.zeros((L,), jnp.float32) for _ in range(DV))
        acc = lax.fori_loop(0, R, acc_row, zero, unroll=4)
        for j in range(DV):
            sums_v[pl.ds(j * L, L)] = acc[j]

        def l1_row(k, carry):
            mv = plsc.load_gather(sums_v, [_splat(k)])
            return tuple(carry[j] + mv * w1t_v[k, pl.ds(j * L, L)]
                         for j in range(DV))

        h1 = lax.fori_loop(0, D, l1_row, zero, unroll=4)
        for j in range(DV):
            h_v[pl.ds(j * L, L)] = jnp.maximum(
                h1[j] + b1_v[pl.ds(j * L, L)], 0.0)

        def l2_row(k, carry):
            hv = plsc.load_gather(h_v, [_splat(k)])
            return tuple(carry[j] + hv * w2t_v[k, pl.ds(j * L, L)]
                         for j in range(DV))

        h2 = lax.fori_loop(0, D, l2_row, zero, unroll=4)
        ratio = br_v[...]
        one_m_ratio = 1.0 - ratio
        col = lax.broadcasted_iota(jnp.int32, (L,), 0)
        gidx = plsc.load_gather(gid_v, [_splat(b)])
        te = []
        for j in range(DV):
            infer = h2[j] + b2_v[pl.ds(j * L, L)]
            known = plsc.load_gather(known_v, [gidx, col + j * L])
            te.append((known * one_m_ratio + infer) * ratio)

        @pl.when(b + K < BPW)
        def _():
            nxt = b + K

            @pl.when(nxt >= NRING)
            def _():
                prev = nxt - NRING
                pltpu.make_async_copy(
                    rows_v.at[prev % NRING],
                    out_hbm.at[pl.ds((w * BPW + prev) * R, R)],
                    ssem.at[prev % NRING]).wait()

            fire(nxt)

        def add_row(r, carry):
            for j in range(DV):
                rows_v[buf, r, pl.ds(j * L, L)] = (
                    rows_v[buf, r, pl.ds(j * L, L)] + te[j])
            return carry

        lax.fori_loop(0, R, add_row, 0, unroll=4)
        pltpu.async_copy(rows_v.at[buf],
                         out_hbm.at[pl.ds((w * BPW + b) * R, R)],
                         ssem.at[buf])

    for d in range(NRING):
        b = BPW - NRING + d
        pltpu.make_async_copy(rows_v.at[b % NRING],
                              out_hbm.at[pl.ds((w * BPW + b) * R, R)],
                              ssem.at[b % NRING]).wait()


def kernel(obs_tokens, game_ids, token_table, known_table, W1, b1, W2, b2,
           blend_ratio):
    Bh, Th, Kh = obs_tokens.shape
    tok = obs_tokens.reshape(NW, NSL, SLICE)
    known_pad = jnp.zeros((64, D), jnp.float32).at[:known_table.shape[0]].set(
        known_table)
    w1t = W1.T * (1.0 / R)
    w2t = W2.T
    br16 = jnp.full((L,), blend_ratio, jnp.float32)

    mesh = plsc.VectorSubcoreMesh(core_axis_name="c", subcore_axis_name="s",
                                  num_cores=NC, num_subcores=NS)

    fused = pl.kernel(
        _body,
        out_type=jax.ShapeDtypeStruct((B * R, D), jnp.float32),
        mesh=mesh,
        compiler_params=pltpu.CompilerParams(needs_layout_passes=False),
        scratch_types=[
            pltpu.VMEM((NSL, SLICE), jnp.int32),
            pltpu.VMEM((NRING, R, D), jnp.float32),
            pltpu.VMEM((D, D), jnp.float32),
            pltpu.VMEM((D, D), jnp.float32),
            pltpu.VMEM((64, D), jnp.float32),
            pltpu.VMEM((D,), jnp.float32),
            pltpu.VMEM((D,), jnp.float32),
            pltpu.VMEM((BPW,), jnp.int32),
            pltpu.VMEM((L,), jnp.float32),
            pltpu.VMEM((D,), jnp.float32),
            pltpu.VMEM((D,), jnp.float32),
            pltpu.SemaphoreType.DMA((NRING,)),
            pltpu.SemaphoreType.DMA((NRING,)),
        ],
    )
    out = fused(tok, game_ids, token_table, known_pad, w1t, b1, w2t, b2,
                br16)
    return out.reshape(Bh, Th * Kh, D)


# speedup vs baseline: 1.0044x; 1.0044x over previous
import jax
import jax.numpy as jnp
from jax import lax
from jax.experimental import pallas as pl
from jax.experimental.pallas import tpu as pltpu
from jax.experimental.pallas import tpu_sc as plsc

NC = 2
NS = 16
NW = NC * NS
L = 16

B = 1024
R = 160
D = 128
BPW = B // NW
SLICE = 80
SPB = R // SLICE
NSL = BPW * SPB
DV = D // L
NRING = 3
K = 2


def _splat(x):
    return jnp.full((L,), x, jnp.int32)


def _body(tok_hbm, gid_hbm, table_hbm, known_hbm, w1t_hbm, b1_hbm, w2t_hbm,
          b2_hbm, br_hbm, out_hbm,
          idx_v, rows_v, w1t_v, w2t_v, known_v, b1_v, b2_v, gid_v,
          br_v, sums_v, h_v, gsem, ssem):
    w = lax.axis_index("s") * NC + lax.axis_index("c")
    pltpu.sync_copy(tok_hbm.at[w], idx_v)
    pltpu.sync_copy(gid_hbm.at[pl.ds(w * BPW, BPW)], gid_v)
    pltpu.sync_copy(known_hbm, known_v)
    pltpu.sync_copy(w1t_hbm, w1t_v)
    pltpu.sync_copy(w2t_hbm, w2t_v)
    pltpu.sync_copy(b1_hbm, b1_v)
    pltpu.sync_copy(b2_hbm, b2_v)
    pltpu.sync_copy(br_hbm, br_v)

    def fire(b):
        buf = b % NRING
        for h in range(SPB):
            pltpu.async_copy(table_hbm.at[idx_v.at[SPB * b + h]],
                             rows_v.at[buf, pl.ds(h * SLICE, SLICE)],
                             gsem.at[buf])

    for b in range(K):
        fire(b)

    @pl.loop(0, BPW)
    def _per_b(b):
        buf = b % NRING
        for h in range(SPB):
            pltpu.make_async_copy(table_hbm.at[idx_v.at[SPB * b + h]],
                                  rows_v.at[buf, pl.ds(h * SLICE, SLICE)],
                                  gsem.at[buf]).wait()

        def acc_row(r, carry):
            return tuple(carry[j] + rows_v[buf, r, pl.ds(j * L, L)]
                         for j in range(DV))

        zero = tuple(jnp.zeros((L,), jnp.float32) for _ in range(DV))
        acc = lax.fori_loop(0, R, acc_row, zero, unroll=4)
        for j in range(DV):
            sums_v[pl.ds(j * L, L)] = acc[j]

        def matvec_row(wref, sref, k, carry):
            mv = plsc.load_gather(sref, [_splat(k)])
            out = list(carry)
            for c in range(DV // 2):
                wi = wref[pl.ds(k * (D // 2) + c * L, L)]
                wv = plsc.bitcast(wi, jnp.bfloat16)
                lo, hi = plsc.unpack(wv, format=plsc.PackFormat.INTERLEAVED)
                out[2 * c] = out[2 * c] + mv * lo
                out[2 * c + 1] = out[2 * c + 1] + mv * hi
            return tuple(out)

        h1 = lax.fori_loop(
            0, D, lambda k, c: matvec_row(w1t_v, sums_v, k, c), zero,
            unroll=4)
        for j in range(DV):
            h_v[pl.ds(j * L, L)] = jnp.maximum(
                h1[j] + b1_v[pl.ds(j * L, L)], 0.0)

        h2 = lax.fori_loop(
            0, D, lambda k, c: matvec_row(w2t_v, h_v, k, c), zero,
            unroll=4)
        ratio = br_v[...]
        one_m_ratio = 1.0 - ratio
        col = lax.broadcasted_iota(jnp.int32, (L,), 0)
        gidx = plsc.load_gather(gid_v, [_splat(b)])
        te = []
        for j in range(DV):
            infer = h2[j] + b2_v[pl.ds(j * L, L)]
            known = plsc.load_gather(known_v, [gidx, col + j * L])
            te.append((known * one_m_ratio + infer) * ratio)

        @pl.when(b + K < BPW)
        def _():
            nxt = b + K

            @pl.when(nxt >= NRING)
            def _():
                prev = nxt - NRING
                pltpu.make_async_copy(
                    rows_v.at[prev % NRING],
                    out_hbm.at[pl.ds((w * BPW + prev) * R, R)],
                    ssem.at[prev % NRING]).wait()

            fire(nxt)

        def add_row(r, carry):
            for j in range(DV):
                rows_v[buf, r, pl.ds(j * L, L)] = (
                    rows_v[buf, r, pl.ds(j * L, L)] + te[j])
            return carry

        lax.fori_loop(0, R, add_row, 0, unroll=4)
        pltpu.async_copy(rows_v.at[buf],
                         out_hbm.at[pl.ds((w * BPW + b) * R, R)],
                         ssem.at[buf])

    for d in range(NRING):
        b = BPW - NRING + d
        pltpu.make_async_copy(rows_v.at[b % NRING],
                              out_hbm.at[pl.ds((w * BPW + b) * R, R)],
                              ssem.at[b % NRING]).wait()


def kernel(obs_tokens, game_ids, token_table, known_table, W1, b1, W2, b2,
           blend_ratio):
    Bh, Th, Kh = obs_tokens.shape
    tok = obs_tokens.reshape(NW, NSL, SLICE)
    known_pad = jnp.zeros((64, D), jnp.float32).at[:known_table.shape[0]].set(
        known_table)

    def _ileave(wt):
        ilv = (wt.reshape(D, DV // 2, 2, L).transpose(0, 1, 3, 2)
               .reshape(D * D).astype(jnp.bfloat16))
        return lax.bitcast_convert_type(ilv.reshape(D * D // 2, 2),
                                        jnp.int32)

    w1t = _ileave(W1.T * (1.0 / R))
    w2t = _ileave(W2.T)
    br16 = jnp.full((L,), blend_ratio, jnp.float32)

    mesh = plsc.VectorSubcoreMesh(core_axis_name="c", subcore_axis_name="s",
                                  num_cores=NC, num_subcores=NS)

    fused = pl.kernel(
        _body,
        out_type=jax.ShapeDtypeStruct((B * R, D), jnp.float32),
        mesh=mesh,
        compiler_params=pltpu.CompilerParams(needs_layout_passes=False),
        scratch_types=[
            pltpu.VMEM((NSL, SLICE), jnp.int32),
            pltpu.VMEM((NRING, R, D), jnp.float32),
            pltpu.VMEM((D * D // 2,), jnp.int32),
            pltpu.VMEM((D * D // 2,), jnp.int32),
            pltpu.VMEM((64, D), jnp.float32),
            pltpu.VMEM((D,), jnp.float32),
            pltpu.VMEM((D,), jnp.float32),
            pltpu.VMEM((BPW,), jnp.int32),
            pltpu.VMEM((L,), jnp.float32),
            pltpu.VMEM((D,), jnp.float32),
            pltpu.VMEM((D,), jnp.float32),
            pltpu.SemaphoreType.DMA((NRING,)),
            pltpu.SemaphoreType.DMA((NRING,)),
        ],
    )
    out = fused(tok, game_ids, token_table, known_pad, w1t, b1, w2t, b2,
                br16)
    return out.reshape(Bh, Th * Kh, D)


# speedup vs baseline: 1.0055x; 1.0011x over previous
import jax
import jax.numpy as jnp
from jax import lax
from jax.experimental import pallas as pl
from jax.experimental.pallas import tpu as pltpu
from jax.experimental.pallas import tpu_sc as plsc

NC = 2
NS = 16
NW = NC * NS
L = 16

B = 1024
R = 160
D = 128
BPW = B // NW
SLICE = 80
SPB = R // SLICE
NSL = BPW * SPB
DV = D // L
NRING = 3
K = 2


def _splat(x):
    return jnp.full((L,), x, jnp.int32)


def _body(tok_hbm, gid_hbm, table_hbm, known_hbm, w1t_hbm, b1_hbm, w2t_hbm,
          b2_hbm, br_hbm, out_hbm,
          idx_v, rows_v, w1t_v, w2t_v, known_v, b1_v, b2_v, gid_v,
          br_v, sums_v, h_v, gsem, ssem):
    w = lax.axis_index("s") * NC + lax.axis_index("c")
    pltpu.sync_copy(tok_hbm.at[w], idx_v)
    pltpu.sync_copy(gid_hbm.at[pl.ds(w * BPW, BPW)], gid_v)
    pltpu.sync_copy(known_hbm, known_v)
    pltpu.sync_copy(w1t_hbm, w1t_v)
    pltpu.sync_copy(w2t_hbm, w2t_v)
    pltpu.sync_copy(b1_hbm, b1_v)
    pltpu.sync_copy(b2_hbm, b2_v)
    pltpu.sync_copy(br_hbm, br_v)

    def fire(b):
        buf = b % NRING
        for h in range(SPB):
            pltpu.async_copy(table_hbm.at[idx_v.at[SPB * b + h]],
                             rows_v.at[buf, pl.ds(h * SLICE, SLICE)],
                             gsem.at[buf])

    for b in range(K):
        fire(b)

    @pl.loop(0, BPW)
    def _per_b(b):
        buf = b % NRING
        for h in range(SPB):
            pltpu.make_async_copy(table_hbm.at[idx_v.at[SPB * b + h]],
                                  rows_v.at[buf, pl.ds(h * SLICE, SLICE)],
                                  gsem.at[buf]).wait()

        zero = tuple(jnp.zeros((L,), jnp.float32) for _ in range(DV))

        def acc_row(r, carry):
            return tuple(carry[j] + rows_v[buf, r, pl.ds(j * L, L)]
                         for j in range(DV))

        acc = plsc.parallel_loop(0, R, unroll=4, carry=zero)(acc_row)
        for j in range(DV):
            sums_v[pl.ds(j * L, L)] = acc[j]

        def matvec_row(wref, sref, k, carry):
            mv = plsc.load_gather(sref, [_splat(k)])
            out = list(carry)
            for c in range(DV // 2):
                wi = wref[pl.ds(k * (D // 2) + c * L, L)]
                wv = plsc.bitcast(wi, jnp.bfloat16)
                lo, hi = plsc.unpack(wv, format=plsc.PackFormat.INTERLEAVED)
                out[2 * c] = out[2 * c] + mv * lo
                out[2 * c + 1] = out[2 * c + 1] + mv * hi
            return tuple(out)

        h1 = plsc.parallel_loop(0, D, unroll=4, carry=zero)(
            lambda k, c: matvec_row(w1t_v, sums_v, k, c))
        for j in range(DV):
            h_v[pl.ds(j * L, L)] = jnp.maximum(
                h1[j] + b1_v[pl.ds(j * L, L)], 0.0)

        h2 = plsc.parallel_loop(0, D, unroll=4, carry=zero)(
            lambda k, c: matvec_row(w2t_v, h_v, k, c))
        ratio = br_v[...]
        one_m_ratio = 1.0 - ratio
        col = lax.broadcasted_iota(jnp.int32, (L,), 0)
        gidx = plsc.load_gather(gid_v, [_splat(b)])
        te = []
        for j in range(DV):
            infer = h2[j] + b2_v[pl.ds(j * L, L)]
            known = plsc.load_gather(known_v, [gidx, col + j * L])
            te.append((known * one_m_ratio + infer) * ratio)

        @pl.when(b + K < BPW)
        def _():
            nxt = b + K

            @pl.when(nxt >= NRING)
            def _():
                prev = nxt - NRING
                pltpu.make_async_copy(
                    rows_v.at[prev % NRING],
                    out_hbm.at[pl.ds((w * BPW + prev) * R, R)],
                    ssem.at[prev % NRING]).wait()

            fire(nxt)

        @plsc.parallel_loop(0, R, unroll=4)
        def _add_row(r):
            for j in range(DV):
                rows_v[buf, r, pl.ds(j * L, L)] = (
                    rows_v[buf, r, pl.ds(j * L, L)] + te[j])
        pltpu.async_copy(rows_v.at[buf],
                         out_hbm.at[pl.ds((w * BPW + b) * R, R)],
                         ssem.at[buf])

    for d in range(NRING):
        b = BPW - NRING + d
        pltpu.make_async_copy(rows_v.at[b % NRING],
                              out_hbm.at[pl.ds((w * BPW + b) * R, R)],
                              ssem.at[b % NRING]).wait()


def kernel(obs_tokens, game_ids, token_table, known_table, W1, b1, W2, b2,
           blend_ratio):
    Bh, Th, Kh = obs_tokens.shape
    tok = obs_tokens.reshape(NW, NSL, SLICE)
    known_pad = jnp.zeros((64, D), jnp.float32).at[:known_table.shape[0]].set(
        known_table)

    def _ileave(wt):
        ilv = (wt.reshape(D, DV // 2, 2, L).transpose(0, 1, 3, 2)
               .reshape(D * D).astype(jnp.bfloat16))
        return lax.bitcast_convert_type(ilv.reshape(D * D // 2, 2),
                                        jnp.int32)

    w1t = _ileave(W1.T * (1.0 / R))
    w2t = _ileave(W2.T)
    br16 = jnp.full((L,), blend_ratio, jnp.float32)

    mesh = plsc.VectorSubcoreMesh(core_axis_name="c", subcore_axis_name="s",
                                  num_cores=NC, num_subcores=NS)

    fused = pl.kernel(
        _body,
        out_type=jax.ShapeDtypeStruct((B * R, D), jnp.float32),
        mesh=mesh,
        compiler_params=pltpu.CompilerParams(needs_layout_passes=False),
        scratch_types=[
            pltpu.VMEM((NSL, SLICE), jnp.int32),
            pltpu.VMEM((NRING, R, D), jnp.float32),
            pltpu.VMEM((D * D // 2,), jnp.int32),
            pltpu.VMEM((D * D // 2,), jnp.int32),
            pltpu.VMEM((64, D), jnp.float32),
            pltpu.VMEM((D,), jnp.float32),
            pltpu.VMEM((D,), jnp.float32),
            pltpu.VMEM((BPW,), jnp.int32),
            pltpu.VMEM((L,), jnp.float32),
            pltpu.VMEM((D,), jnp.float32),
            pltpu.VMEM((D,), jnp.float32),
            pltpu.SemaphoreType.DMA((NRING,)),
            pltpu.SemaphoreType.DMA((NRING,)),
        ],
    )
    out = fused(tok, game_ids, token_table, known_pad, w1t, b1, w2t, b2,
                br16)
    return out.reshape(Bh, Th * Kh, D)


# speedup vs baseline: 1.0091x; 1.0035x over previous
import jax
import jax.numpy as jnp
from jax import lax
from jax.experimental import pallas as pl
from jax.experimental.pallas import tpu as pltpu
from jax.experimental.pallas import tpu_sc as plsc

NC = 2
NS = 16
NW = NC * NS
L = 16

B = 1024
R = 160
D = 128
BPW = B // NW
SLICE = 80
SPB = R // SLICE
NSL = BPW * SPB
DV = D // L
NRING = 4
K = 2


def _splat(x):
    return jnp.full((L,), x, jnp.int32)


def _body(tok_hbm, gid_hbm, table_hbm, known_hbm, w1t_hbm, b1_hbm, w2t_hbm,
          b2_hbm, br_hbm, out_hbm,
          idx_v, rows_v, w1t_v, w2t_v, known_v, b1_v, b2_v, gid_v,
          br_v, sums_v, h_v, gsem, ssem):
    w = lax.axis_index("s") * NC + lax.axis_index("c")
    pltpu.sync_copy(tok_hbm.at[w], idx_v)
    pltpu.sync_copy(gid_hbm.at[pl.ds(w * BPW, BPW)], gid_v)
    pltpu.sync_copy(known_hbm, known_v)
    pltpu.sync_copy(w1t_hbm, w1t_v)
    pltpu.sync_copy(w2t_hbm, w2t_v)
    pltpu.sync_copy(b1_hbm, b1_v)
    pltpu.sync_copy(b2_hbm, b2_v)
    pltpu.sync_copy(br_hbm, br_v)

    def fire(b):
        buf = b % NRING
        for h in range(SPB):
            pltpu.async_copy(table_hbm.at[idx_v.at[SPB * b + h]],
                             rows_v.at[buf, pl.ds(h * SLICE, SLICE)],
                             gsem.at[buf])

    for b in range(K):
        fire(b)

    @pl.loop(0, BPW)
    def _per_b(b):
        buf = b % NRING
        for h in range(SPB):
            pltpu.make_async_copy(table_hbm.at[idx_v.at[SPB * b + h]],
                                  rows_v.at[buf, pl.ds(h * SLICE, SLICE)],
                                  gsem.at[buf]).wait()

        zero = tuple(jnp.zeros((L,), jnp.float32) for _ in range(DV))

        def acc_row(r, carry):
            return tuple(carry[j] + rows_v[buf, r, pl.ds(j * L, L)]
                         for j in range(DV))

        acc = plsc.parallel_loop(0, R, unroll=4, carry=zero)(acc_row)
        for j in range(DV):
            sums_v[pl.ds(j * L, L)] = acc[j]

        def matvec_row(wref, sref, k, carry):
            mv = plsc.load_gather(sref, [_splat(k)])
            out = list(carry)
            for c in range(DV // 2):
                wi = wref[pl.ds(k * (D // 2) + c * L, L)]
                wv = plsc.bitcast(wi, jnp.bfloat16)
                lo, hi = plsc.unpack(wv, format=plsc.PackFormat.INTERLEAVED)
                out[2 * c] = out[2 * c] + mv * lo
                out[2 * c + 1] = out[2 * c + 1] + mv * hi
            return tuple(out)

        h1 = plsc.parallel_loop(0, D, unroll=4, carry=zero)(
            lambda k, c: matvec_row(w1t_v, sums_v, k, c))
        for j in range(DV):
            h_v[pl.ds(j * L, L)] = jnp.maximum(
                h1[j] + b1_v[pl.ds(j * L, L)], 0.0)

        h2 = plsc.parallel_loop(0, D, unroll=4, carry=zero)(
            lambda k, c: matvec_row(w2t_v, h_v, k, c))
        ratio = br_v[...]
        one_m_ratio = 1.0 - ratio
        col = lax.broadcasted_iota(jnp.int32, (L,), 0)
        gidx = plsc.load_gather(gid_v, [_splat(b)])
        te = []
        for j in range(DV):
            infer = h2[j] + b2_v[pl.ds(j * L, L)]
            known = plsc.load_gather(known_v, [gidx, col + j * L])
            te.append((known * one_m_ratio + infer) * ratio)

        @pl.when(b + K < BPW)
        def _():
            nxt = b + K

            @pl.when(nxt >= NRING)
            def _():
                prev = nxt - NRING
                pltpu.make_async_copy(
                    rows_v.at[prev % NRING],
                    out_hbm.at[pl.ds((w * BPW + prev) * R, R)],
                    ssem.at[prev % NRING]).wait()

            fire(nxt)

        @plsc.parallel_loop(0, R, unroll=4)
        def _add_row(r):
            for j in range(DV):
                rows_v[buf, r, pl.ds(j * L, L)] = (
                    rows_v[buf, r, pl.ds(j * L, L)] + te[j])
        pltpu.async_copy(rows_v.at[buf],
                         out_hbm.at[pl.ds((w * BPW + b) * R, R)],
                         ssem.at[buf])

    for d in range(NRING):
        b = BPW - NRING + d
        pltpu.make_async_copy(rows_v.at[b % NRING],
                              out_hbm.at[pl.ds((w * BPW + b) * R, R)],
                              ssem.at[b % NRING]).wait()


def kernel(obs_tokens, game_ids, token_table, known_table, W1, b1, W2, b2,
           blend_ratio):
    Bh, Th, Kh = obs_tokens.shape
    tok = obs_tokens.reshape(NW, NSL, SLICE)
    known_pad = jnp.zeros((64, D), jnp.float32).at[:known_table.shape[0]].set(
        known_table)

    def _ileave(wt):
        ilv = (wt.reshape(D, DV // 2, 2, L).transpose(0, 1, 3, 2)
               .reshape(D * D).astype(jnp.bfloat16))
        return lax.bitcast_convert_type(ilv.reshape(D * D // 2, 2),
                                        jnp.int32)

    w1t = _ileave(W1.T * (1.0 / R))
    w2t = _ileave(W2.T)
    br16 = jnp.full((L,), blend_ratio, jnp.float32)

    mesh = plsc.VectorSubcoreMesh(core_axis_name="c", subcore_axis_name="s",
                                  num_cores=NC, num_subcores=NS)

    fused = pl.kernel(
        _body,
        out_type=jax.ShapeDtypeStruct((B * R, D), jnp.float32),
        mesh=mesh,
        compiler_params=pltpu.CompilerParams(needs_layout_passes=False),
        scratch_types=[
            pltpu.VMEM((NSL, SLICE), jnp.int32),
            pltpu.VMEM((NRING, R, D), jnp.float32),
            pltpu.VMEM((D * D // 2,), jnp.int32),
            pltpu.VMEM((D * D // 2,), jnp.int32),
            pltpu.VMEM((64, D), jnp.float32),
            pltpu.VMEM((D,), jnp.float32),
            pltpu.VMEM((D,), jnp.float32),
            pltpu.VMEM((BPW,), jnp.int32),
            pltpu.VMEM((L,), jnp.float32),
            pltpu.VMEM((D,), jnp.float32),
            pltpu.VMEM((D,), jnp.float32),
            pltpu.SemaphoreType.DMA((NRING,)),
            pltpu.SemaphoreType.DMA((NRING,)),
        ],
    )
    out = fused(tok, game_ids, token_table, known_pad, w1t, b1, w2t, b2,
                br16)
    return out.reshape(Bh, Th * Kh, D)
